# Initial kernel scaffold; baseline (speedup 1.0000x reference)
import jax
import jax.numpy as jnp
from jax.experimental import pallas as pl


def kernel(x):
    raise NotImplementedError("write your pallas kernel here")


# speedup vs baseline: 7.4147x; 7.4147x over previous
import functools

import jax
import jax.numpy as jnp
from jax.experimental import pallas as pl

NUM_NEIGHBORS = 20
ROWS_PER_BLOCK = 256


def _knn_block_kernel(xr_ref, xc_ref, out_ref, *, n_points, k):
    xr = xr_ref[0]
    xc = xc_ref[0]
    xr0 = xr[:, 0:1]
    xr1 = xr[:, 1:2]
    xr2 = xr[:, 2:3]
    xc0 = xc[0:1, :]
    xc1 = xc[1:2, :]
    xc2 = xc[2:3, :]
    x2r = xr0 * xr0 + xr1 * xr1 + xr2 * xr2
    x2c = xc0 * xc0 + xc1 * xc1 + xc2 * xc2
    def _b(v):
        return v.astype(jnp.bfloat16).astype(jnp.float32)
    dot = _b(xr0) * _b(xc0) + _b(xr1) * _b(xc1) + _b(xr2) * _b(xc2)
    d = (x2r + x2c) - 2.0 * dot

    iota = jax.lax.broadcasted_iota(jnp.int32, d.shape, 1)
    inf = jnp.float32(jnp.inf)
    big = jnp.int32(n_points)
    cols = []
    for _ in range(k):
        m = jnp.min(d, axis=1, keepdims=True)
        idx = jnp.min(jnp.where(d == m, iota, big), axis=1, keepdims=True)
        cols.append(idx)
        d = jnp.where(iota == idx, inf, d)
    out_ref[0] = jnp.concatenate(cols, axis=1)


def _knn_topk_indices(x):
    n_samples, n_points, _ = x.shape
    rows = ROWS_PER_BLOCK
    k = NUM_NEIGHBORS
    xr = jnp.pad(x, ((0, 0), (0, 0), (0, 1)))
    xc = jnp.pad(jnp.swapaxes(x, 1, 2), ((0, 0), (0, 5), (0, 0)))
    grid = (n_samples, n_points // rows)
    return pl.pallas_call(
        functools.partial(_knn_block_kernel, n_points=n_points, k=k),
        grid=grid,
        in_specs=[
            pl.BlockSpec((1, rows, 4), lambda s, r: (s, r, 0)),
            pl.BlockSpec((1, 8, n_points), lambda s, r: (s, 0, 0)),
        ],
        out_specs=pl.BlockSpec((1, rows, k), lambda s, r: (s, r, 0)),
        out_shape=jax.ShapeDtypeStruct((n_samples, n_points, k), jnp.int32),
    )(xr, xc)


def kernel(x):
    if x.ndim == 2:
        x = x[None, :, :]
    n_samples, n_points, _ = x.shape
    k_indices = _knn_topk_indices(x)
    dst = k_indices.astype(jnp.int64)
    src = jnp.zeros_like(dst) + jnp.arange(n_points, dtype=jnp.int64).reshape(1, -1, 1)
    per_sample_offset = (jnp.arange(n_samples, dtype=jnp.int64) * n_points).reshape(-1, 1, 1)
    dst = dst + per_sample_offset
    src = src + per_sample_offset
    return src.reshape(-1), dst.reshape(-1)
